# baseline (speedup 1.0000x reference)
import functools

import jax
import jax.numpy as jnp
from jax.experimental import pallas as pl
from jax.experimental.pallas import tpu as pltpu
from jax.experimental.pallas import tpu_sc as plsc

N_TOK = 8192
N_CODE = 8192
D = 32
TM = 1024
TN = 512
X_NUMEL = 8 * 32 * 32 * 32


def _argmin_loss_kernel(x_ref, e_ref, idx_ref, loss_ref, esq_ref):
    i = pl.program_id(0)

    @pl.when(i == 0)
    def _():
        e = e_ref[...]
        esq_ref[...] = jnp.sum(e * e, axis=1).reshape(1, N_CODE)

    x = x_ref[...]
    xsq = jnp.sum(x * x, axis=1, keepdims=True)
    xm2 = x * (-2.0)
    lane = jax.lax.broadcasted_iota(
        jnp.int32, (TM, TN), 1).astype(jnp.float32)
    big = jnp.float32(N_CODE)

    def body(j, carry):
        best_val, best_idx = carry
        e = e_ref[pl.ds(j * TN, TN), :]
        esq = esq_ref[:, pl.ds(j * TN, TN)]
        mm2 = jax.lax.dot_general(
            xm2, e, (((1,), (1,)), ((), ())),
            preferred_element_type=jnp.float32)
        scores = (xsq + esq) + mm2
        tile_min = jnp.min(scores, axis=1, keepdims=True)
        tile_idx = (jnp.argmin(scores, axis=1)[:, None].astype(jnp.float32)
                    + jnp.float32(j * TN))
        take = tile_min < best_val
        return (jnp.where(take, tile_min, best_val),
                jnp.where(take, tile_idx, best_idx))

    init = (jnp.full((TM, 1), jnp.inf, jnp.float32),
            jnp.zeros((TM, 1), jnp.float32))
    best_val, best_idx = jax.lax.fori_loop(0, N_CODE // TN, body, init,
                                           unroll=8)
    idx_ref[...] = best_idx.astype(jnp.int32)

    part = jnp.sum(best_val)
    prev = jnp.where(i == 0, 0.0, loss_ref[0, 0])
    acc = prev + part
    n_tiles = N_TOK // TM
    loss_ref[0, 0] = jnp.where(i == n_tiles - 1,
                               acc * (2.0 / X_NUMEL), acc)


def _argmin_loss(flat_x, embedding):
    return pl.pallas_call(
        _argmin_loss_kernel,
        grid=(N_TOK // TM,),
        in_specs=[
            pl.BlockSpec((TM, D), lambda i: (i, 0)),
            pl.BlockSpec((N_CODE, D), lambda i: (0, 0)),
        ],
        out_specs=[
            pl.BlockSpec((TM, 1), lambda i: (i, 0)),
            pl.BlockSpec(block_shape=(1, 1), index_map=lambda i: (0, 0),
                         memory_space=pltpu.SMEM),
        ],
        out_shape=[
            jax.ShapeDtypeStruct((N_TOK, 1), jnp.int32),
            jax.ShapeDtypeStruct((1, 1), jnp.float32),
        ],
        scratch_shapes=[pltpu.VMEM((1, N_CODE), jnp.float32)],
    )(flat_x, embedding)


GATHER_W = 128
GATHER_D = 128


def _sc_gather(embedding_padded, idxs_row):
    mesh = plsc.VectorSubcoreMesh(core_axis_name="core",
                                  subcore_axis_name="subcore")

    @functools.partial(
        pl.kernel,
        out_type=jax.ShapeDtypeStruct((N_TOK, GATHER_D), jnp.float32),
        mesh=mesh)
    def kern(e_hbm, i_hbm, o_hbm):
        def body(i_vmem, o_vmem):
            pltpu.sync_copy(e_hbm.at[i_vmem.at[0]], o_vmem)

        pltpu.emit_pipeline(
            body,
            grid=(N_TOK // GATHER_W,),
            in_specs=[pl.BlockSpec((1, GATHER_W), index_map=lambda i: (0, i))],
            out_specs=[pl.BlockSpec((GATHER_W, GATHER_D),
                                    index_map=lambda i: (i, 0))],
            core_axis_name=("core", "subcore"),
            dimension_semantics=(pltpu.PARALLEL,),
        )(i_hbm, o_hbm)

    return kern(embedding_padded, idxs_row)


def kernel(x, embedding):
    n, c, h, w = x.shape
    flat_x = jnp.transpose(x, (0, 2, 3, 1)).reshape(-1, c)
    idxs, loss = _argmin_loss(flat_x, embedding)
    e_pad = jnp.pad(embedding, ((0, 0), (0, GATHER_D - D)))
    quantized = _sc_gather(e_pad, idxs.reshape(1, N_TOK))[:, :D]
    quantized = jnp.transpose(quantized.reshape(n, h, w, c), (0, 3, 1, 2))
    return quantized, loss.reshape(())


# speedup vs baseline: 2.1212x; 2.1212x over previous
import functools

import jax
import jax.numpy as jnp
from jax.experimental import pallas as pl
from jax.experimental.pallas import tpu as pltpu
from jax.experimental.pallas import tpu_sc as plsc

N_TOK = 8192
N_CODE = 8192
D = 32
TM = 1024
TN = 1024
X_NUMEL = 8 * 32 * 32 * 32


def _argmin_loss_kernel(x_ref, e_ref, idx_ref, loss_ref, esq_ref):
    i = pl.program_id(0)

    @pl.when(i == 0)
    def _():
        e = e_ref[...]
        esq_ref[...] = jnp.sum(e * e, axis=1).reshape(1, N_CODE)

    x = x_ref[...]
    xsq = jnp.sum(x * x, axis=1, keepdims=True)
    xm2 = x * (-2.0)
    lane = jax.lax.broadcasted_iota(
        jnp.int32, (TM, TN), 1).astype(jnp.float32)
    big = jnp.float32(N_CODE)

    def body(j, carry):
        best_val, best_idx = carry
        e = e_ref[pl.ds(j * TN, TN), :]
        esq = esq_ref[:, pl.ds(j * TN, TN)]
        mm2 = jax.lax.dot_general(
            xm2, e, (((1,), (1,)), ((), ())),
            preferred_element_type=jnp.float32)
        scores = (xsq + esq) + mm2
        tile_min = jnp.min(scores, axis=1, keepdims=True)
        tile_idx = jnp.min(
            jnp.where(scores == tile_min, lane, big),
            axis=1, keepdims=True) + jnp.float32(j * TN)
        take = tile_min < best_val
        return (jnp.where(take, tile_min, best_val),
                jnp.where(take, tile_idx, best_idx))

    init = (jnp.full((TM, 1), jnp.inf, jnp.float32),
            jnp.zeros((TM, 1), jnp.float32))
    best_val, best_idx = jax.lax.fori_loop(0, N_CODE // TN, body, init,
                                           unroll=8)
    idx_ref[...] = best_idx.astype(jnp.int32)

    part = jnp.sum(best_val)
    prev = jnp.where(i == 0, 0.0, loss_ref[0, 0])
    acc = prev + part
    n_tiles = N_TOK // TM
    loss_ref[0, 0] = jnp.where(i == n_tiles - 1,
                               acc * (2.0 / X_NUMEL), acc)


def _argmin_loss(flat_x, embedding):
    return pl.pallas_call(
        _argmin_loss_kernel,
        grid=(N_TOK // TM,),
        in_specs=[
            pl.BlockSpec((TM, D), lambda i: (i, 0)),
            pl.BlockSpec((N_CODE, D), lambda i: (0, 0)),
        ],
        out_specs=[
            pl.BlockSpec((TM, 1), lambda i: (i, 0)),
            pl.BlockSpec(block_shape=(1, 1), index_map=lambda i: (0, 0),
                         memory_space=pltpu.SMEM),
        ],
        out_shape=[
            jax.ShapeDtypeStruct((N_TOK, 1), jnp.int32),
            jax.ShapeDtypeStruct((1, 1), jnp.float32),
        ],
        scratch_shapes=[pltpu.VMEM((1, N_CODE), jnp.float32)],
    )(flat_x, embedding)


GATHER_W = 128
GATHER_D = 128


def _sc_gather(embedding_padded, idxs_row):
    mesh = plsc.VectorSubcoreMesh(core_axis_name="core",
                                  subcore_axis_name="subcore")

    @functools.partial(
        pl.kernel,
        out_type=jax.ShapeDtypeStruct((N_TOK, GATHER_D), jnp.float32),
        mesh=mesh)
    def kern(e_hbm, i_hbm, o_hbm):
        def body(i_vmem, o_vmem):
            pltpu.sync_copy(e_hbm.at[i_vmem.at[0]], o_vmem)

        pltpu.emit_pipeline(
            body,
            grid=(N_TOK // GATHER_W,),
            in_specs=[pl.BlockSpec((1, GATHER_W), index_map=lambda i: (0, i))],
            out_specs=[pl.BlockSpec((GATHER_W, GATHER_D),
                                    index_map=lambda i: (i, 0))],
            core_axis_name=("core", "subcore"),
            dimension_semantics=(pltpu.PARALLEL,),
        )(i_hbm, o_hbm)

    return kern(embedding_padded, idxs_row)


def kernel(x, embedding):
    n, c, h, w = x.shape
    flat_x = jnp.transpose(x, (0, 2, 3, 1)).reshape(-1, c)
    idxs, loss = _argmin_loss(flat_x, embedding)
    e_pad = jnp.pad(embedding, ((0, 0), (0, GATHER_D - D)))
    quantized = _sc_gather(e_pad, idxs.reshape(1, N_TOK))[:, :D]
    quantized = jnp.transpose(quantized.reshape(n, h, w, c), (0, 3, 1, 2))
    return quantized, loss.reshape(())
